# baseline (speedup 1.0000x reference)
import jax
import jax.numpy as jnp
import numpy as np
from jax import lax
from jax.experimental import pallas as pl
from jax.experimental.pallas import tpu as pltpu
from jax.experimental.pallas import tpu_sc as plsc

BATCH = 4
MODEL_DIM = 1024
MAX_LEN = 2048

NC = 2
NS = 16
LANES = 16
NW = NC * NS

B_TOTAL = BATCH * MAX_LEN
T_PER_W = MAX_LEN // NW
PCH = 16
RCH = 8
HPP = PCH // RCH
N_TC = T_PER_W // PCH
N_CHUNKS = N_TC * BATCH * HPP
NB = 6
GLEAD = 4
NQ = MAX_LEN // PCH

_VR = MODEL_DIM // LANES

_OFF_A1 = 0
_OFF_A2 = NQ * MODEL_DIM
_OFF_B1 = 2 * NQ * MODEL_DIM
_OFF_B2 = 2 * NQ * MODEL_DIM + PCH * MODEL_DIM


def _aux_tables_np():
    i = np.arange(MODEL_DIM, dtype=np.float64)
    w = 1.0 / (10000.0 ** (2.0 * i / MODEL_DIM))
    even = (np.arange(MODEL_DIM) % 2) == 0
    th = PCH * np.arange(NQ, dtype=np.float64).reshape(-1, 1) * w
    a1 = np.where(even, np.sin(th), np.cos(th)).astype(np.float32)
    a2 = np.where(even, np.cos(th), -np.sin(th)).astype(np.float32)
    rw = np.arange(PCH, dtype=np.float64).reshape(-1, 1) * w
    b1 = np.cos(rw).astype(np.float32)
    b2 = np.sin(rw).astype(np.float32)
    return np.concatenate(
        [a1.reshape(-1), a2.reshape(-1), b1.reshape(-1), b2.reshape(-1)]
    )


_AUX = _aux_tables_np()


def _sc_body(table_hbm, idx_hbm, aux_hbm, out_hbm, *scratch):
    idx_v = scratch[0]
    rows = scratch[1 : 1 + NB]
    posrec = scratch[1 + NB]
    b1_v = scratch[2 + NB]
    b2_v = scratch[3 + NB]
    a_v = scratch[4 + NB : 6 + NB]
    gsem = scratch[6 + NB : 6 + 2 * NB]
    psem = scratch[6 + 2 * NB : 6 + 3 * NB]
    asem = scratch[6 + 3 * NB : 8 + 3 * NB]
    bsem = scratch[8 + 3 * NB : 10 + 3 * NB]

    wid = lax.axis_index("s") * NC + lax.axis_index("c")
    t0 = wid * T_PER_W

    for b in range(BATCH):
        pltpu.sync_copy(
            idx_hbm.at[pl.ds(b * MAX_LEN + t0, T_PER_W)],
            idx_v.at[pl.ds(b * T_PER_W, T_PER_W)],
        )

    bd1 = pltpu.async_copy(
        aux_hbm.at[pl.ds(_OFF_B1, PCH * MODEL_DIM)], b1_v, bsem[0]
    )
    bd2 = pltpu.async_copy(
        aux_hbm.at[pl.ds(_OFF_B2, PCH * MODEL_DIM)], b2_v, bsem[1]
    )

    def issue_a(tc):
        q = wid * N_TC + tc
        buf = tc % 2
        d1 = pltpu.async_copy(
            aux_hbm.at[pl.ds(_OFF_A1 + q * MODEL_DIM, MODEL_DIM)],
            a_v[buf].at[pl.ds(0, MODEL_DIM)],
            asem[buf],
        )
        d2 = pltpu.async_copy(
            aux_hbm.at[pl.ds(_OFF_A2 + q * MODEL_DIM, MODEL_DIM)],
            a_v[buf].at[pl.ds(MODEL_DIM, MODEL_DIM)],
            asem[buf],
        )
        return (d1, d2)

    a_desc = [None] * N_TC
    a_desc[0] = issue_a(0)
    a_desc[1] = issue_a(1)

    def chunk_coords(n):
        tc = n // (BATCH * HPP)
        b = (n % (BATCH * HPP)) // HPP
        h = n % HPP
        return tc, b, h

    def issue_gather(n):
        tc, b, h = chunk_coords(n)
        return pltpu.async_copy(
            table_hbm.at[
                idx_v.at[pl.ds(b * T_PER_W + tc * PCH + h * RCH, RCH)]
            ],
            rows[n % NB],
            gsem[n % NB],
        )

    gat = [None] * NB
    put = [None] * NB
    for n in range(GLEAD):
        gat[n % NB] = issue_gather(n)

    for c in range(N_CHUNKS):
        j = c % NB
        tc, b, h = chunk_coords(c)

        if c == BATCH * HPP:
            a_desc[2] = issue_a(2)
        if c == 2 * BATCH * HPP:
            a_desc[3] = issue_a(3)

        n = c + GLEAD
        if n < N_CHUNKS:
            if n >= NB:
                put[n % NB].wait()
            gat[n % NB] = issue_gather(n)

        if b == 0 and h == 0:
            if c == 0:
                bd1.wait()
                bd2.wait()
            a_desc[tc][0].wait()
            a_desc[tc][1].wait()
            a_tc = a_v[tc % 2]

            @plsc.parallel_loop(0, _VR, unroll=2)
            def gen_body(jc):
                off = pl.multiple_of(jc << 4, LANES)
                a1 = a_tc[pl.ds(off, LANES)]
                a2 = a_tc[pl.ds(MODEL_DIM + off, LANES)]
                for r in range(PCH):
                    posrec[r, pl.ds(off, LANES)] = (
                        a1 * b1_v[pl.ds(r * MODEL_DIM + off, LANES)]
                        + a2 * b2_v[pl.ds(r * MODEL_DIM + off, LANES)]
                    )

        gat[j].wait()
        rows_j = rows[j]
        prow0 = h * RCH

        @plsc.parallel_loop(0, RCH * _VR, unroll=8)
        def add_body(i):
            r = i >> 6
            off = pl.multiple_of((i & (_VR - 1)) << 4, LANES)
            rows_j[r, pl.ds(off, LANES)] = (
                rows_j[r, pl.ds(off, LANES)] + posrec[prow0 + r, pl.ds(off, LANES)]
            )

        put[j] = pltpu.async_copy(
            rows_j,
            out_hbm.at[pl.ds(b * MAX_LEN + t0 + tc * PCH + h * RCH, RCH)],
            psem[j],
        )

    for c in range(N_CHUNKS - NB, N_CHUNKS):
        put[c % NB].wait()


@jax.jit
def _embed(idx, table, aux):
    mesh = plsc.VectorSubcoreMesh(
        core_axis_name="c", subcore_axis_name="s", num_cores=NC, num_subcores=NS
    )
    scratch = (
        [pltpu.VMEM((BATCH * T_PER_W,), jnp.int32)]
        + [pltpu.VMEM((RCH, MODEL_DIM), jnp.float32) for _ in range(NB)]
        + [pltpu.VMEM((PCH, MODEL_DIM), jnp.float32)]
        + [pltpu.VMEM((PCH * MODEL_DIM,), jnp.float32)]
        + [pltpu.VMEM((PCH * MODEL_DIM,), jnp.float32)]
        + [pltpu.VMEM((2 * MODEL_DIM,), jnp.float32) for _ in range(2)]
        + [pltpu.SemaphoreType.DMA for _ in range(2 * NB + 4)]
    )
    fn = pl.kernel(
        _sc_body,
        out_type=jax.ShapeDtypeStruct((B_TOTAL, MODEL_DIM), jnp.float32),
        mesh=mesh,
        scratch_types=scratch,
    )
    return fn(table, idx, aux)


def kernel(x, table):
    idx = x.reshape(-1).astype(jnp.int32)
    out = _embed(idx, table, jnp.asarray(_AUX))
    return out.reshape(BATCH, MAX_LEN, MODEL_DIM)


# speedup vs baseline: 1.1037x; 1.1037x over previous
import jax
import jax.numpy as jnp
import numpy as np
from jax import lax
from jax.experimental import pallas as pl
from jax.experimental.pallas import tpu as pltpu
from jax.experimental.pallas import tpu_sc as plsc

BATCH = 4
MODEL_DIM = 1024
MAX_LEN = 2048

NC = 2
NS = 16
LANES = 16
NW = NC * NS

B_TOTAL = BATCH * MAX_LEN
T_PER_W = MAX_LEN // NW
CHUNK = 16
QR = 8
QPC = CHUNK // QR
N_TC = T_PER_W // CHUNK
N_CHUNKS = N_TC * BATCH
NB = 5
GLEAD = 3
NQ = MAX_LEN // QR

_VR = MODEL_DIM // LANES

_OFF_A1 = 0
_OFF_A2 = NQ * MODEL_DIM
_OFF_B1 = 2 * NQ * MODEL_DIM
_OFF_B2 = 2 * NQ * MODEL_DIM + QR * MODEL_DIM


def _aux_tables_np():
    i = np.arange(MODEL_DIM, dtype=np.float64)
    w = 1.0 / (10000.0 ** (2.0 * i / MODEL_DIM))
    even = (np.arange(MODEL_DIM) % 2) == 0
    th = QR * np.arange(NQ, dtype=np.float64).reshape(-1, 1) * w
    a1 = np.where(even, np.sin(th), np.cos(th)).astype(np.float32)
    a2 = np.where(even, np.cos(th), -np.sin(th)).astype(np.float32)
    rw = np.arange(QR, dtype=np.float64).reshape(-1, 1) * w
    b1 = np.cos(rw).astype(np.float32)
    b2 = np.sin(rw).astype(np.float32)
    return np.concatenate(
        [a1.reshape(-1), a2.reshape(-1), b1.reshape(-1), b2.reshape(-1)]
    )


_AUX = _aux_tables_np()


def _sc_body(table_hbm, idx_hbm, aux_hbm, out_hbm, *scratch):
    idx_v = scratch[0]
    rows = scratch[1 : 1 + NB]
    posrec = scratch[1 + NB]
    b1_v = scratch[2 + NB]
    b2_v = scratch[3 + NB]
    a_v = scratch[4 + NB : 6 + NB]
    gsem = scratch[6 + NB : 6 + 2 * NB]
    psem = scratch[6 + 2 * NB : 6 + 3 * NB]
    asem = scratch[6 + 3 * NB : 8 + 3 * NB]
    bsem = scratch[8 + 3 * NB]
    isem = scratch[9 + 3 * NB]

    wid = lax.axis_index("s") * NC + lax.axis_index("c")
    t0 = wid * T_PER_W

    bd1 = pltpu.async_copy(aux_hbm.at[pl.ds(_OFF_B1, QR * MODEL_DIM)], b1_v, bsem)
    bd2 = pltpu.async_copy(aux_hbm.at[pl.ds(_OFF_B2, QR * MODEL_DIM)], b2_v, bsem)

    def issue_a(tc):
        q0 = (wid * N_TC + tc) * QPC
        buf = tc % 2
        d1 = pltpu.async_copy(
            aux_hbm.at[pl.ds(_OFF_A1 + q0 * MODEL_DIM, QPC * MODEL_DIM)],
            a_v[buf].at[pl.ds(0, QPC * MODEL_DIM)],
            asem[buf],
        )
        d2 = pltpu.async_copy(
            aux_hbm.at[pl.ds(_OFF_A2 + q0 * MODEL_DIM, QPC * MODEL_DIM)],
            a_v[buf].at[pl.ds(QPC * MODEL_DIM, QPC * MODEL_DIM)],
            asem[buf],
        )
        return (d1, d2)

    a_desc = [None] * N_TC
    a_desc[0] = issue_a(0)
    a_desc[1] = issue_a(1)

    idx_descs = [
        pltpu.async_copy(
            idx_hbm.at[pl.ds(b * MAX_LEN + t0, T_PER_W)],
            idx_v.at[pl.ds(b * T_PER_W, T_PER_W)],
            isem,
        )
        for b in range(BATCH)
    ]
    for d in idx_descs:
        d.wait()

    def issue_gather(n):
        tc, b = n // BATCH, n % BATCH
        return pltpu.async_copy(
            table_hbm.at[idx_v.at[pl.ds(b * T_PER_W + tc * CHUNK, CHUNK)]],
            rows[n % NB],
            gsem[n % NB],
        )

    gat = [None] * NB
    put = [None] * NB
    for n in range(GLEAD):
        gat[n % NB] = issue_gather(n)

    for c in range(N_CHUNKS):
        j = c % NB
        tc, b = c // BATCH, c % BATCH

        if c == BATCH:
            a_desc[2] = issue_a(2)
        if c == 2 * BATCH:
            a_desc[3] = issue_a(3)

        n = c + GLEAD
        if n < N_CHUNKS:
            if n >= NB:
                put[n % NB].wait()
            gat[n % NB] = issue_gather(n)

        if b == 0:
            if c == 0:
                bd1.wait()
                bd2.wait()
            a_desc[tc][0].wait()
            a_desc[tc][1].wait()
            a_tc = a_v[tc % 2]

            @plsc.parallel_loop(0, _VR, unroll=2)
            def gen_body(jc):
                off = pl.multiple_of(jc << 4, LANES)
                a1q0 = a_tc[pl.ds(off, LANES)]
                a1q1 = a_tc[pl.ds(MODEL_DIM + off, LANES)]
                a2q0 = a_tc[pl.ds(2 * MODEL_DIM + off, LANES)]
                a2q1 = a_tc[pl.ds(3 * MODEL_DIM + off, LANES)]
                for r in range(QR):
                    b1r = b1_v[pl.ds(r * MODEL_DIM + off, LANES)]
                    b2r = b2_v[pl.ds(r * MODEL_DIM + off, LANES)]
                    posrec[r, pl.ds(off, LANES)] = a1q0 * b1r + a2q0 * b2r
                    posrec[QR + r, pl.ds(off, LANES)] = a1q1 * b1r + a2q1 * b2r

        gat[j].wait()
        rows_j = rows[j]

        @plsc.parallel_loop(0, CHUNK * _VR, unroll=8)
        def add_body(i):
            r = i >> 6
            off = pl.multiple_of((i & (_VR - 1)) << 4, LANES)
            rows_j[r, pl.ds(off, LANES)] = (
                rows_j[r, pl.ds(off, LANES)] + posrec[r, pl.ds(off, LANES)]
            )

        put[j] = pltpu.async_copy(
            rows_j,
            out_hbm.at[pl.ds(b * MAX_LEN + t0 + tc * CHUNK, CHUNK)],
            psem[j],
        )

    for c in range(N_CHUNKS - NB, N_CHUNKS):
        put[c % NB].wait()


@jax.jit
def _embed(idx, table, aux):
    mesh = plsc.VectorSubcoreMesh(
        core_axis_name="c", subcore_axis_name="s", num_cores=NC, num_subcores=NS
    )
    scratch = (
        [pltpu.VMEM((BATCH * T_PER_W,), jnp.int32)]
        + [pltpu.VMEM((CHUNK, MODEL_DIM), jnp.float32) for _ in range(NB)]
        + [pltpu.VMEM((CHUNK, MODEL_DIM), jnp.float32)]
        + [pltpu.VMEM((QR * MODEL_DIM,), jnp.float32)]
        + [pltpu.VMEM((QR * MODEL_DIM,), jnp.float32)]
        + [pltpu.VMEM((2 * QPC * MODEL_DIM,), jnp.float32) for _ in range(2)]
        + [pltpu.SemaphoreType.DMA for _ in range(2 * NB + 4)]
    )
    fn = pl.kernel(
        _sc_body,
        out_type=jax.ShapeDtypeStruct((B_TOTAL, MODEL_DIM), jnp.float32),
        mesh=mesh,
        scratch_types=scratch,
    )
    return fn(table, idx, aux)


def kernel(x, table):
    idx = x.reshape(-1).astype(jnp.int32)
    out = _embed(idx, table, jnp.asarray(_AUX))
    return out.reshape(BATCH, MAX_LEN, MODEL_DIM)


# speedup vs baseline: 1.1064x; 1.0025x over previous
import jax
import jax.numpy as jnp
import numpy as np
from jax import lax
from jax.experimental import pallas as pl
from jax.experimental.pallas import tpu as pltpu
from jax.experimental.pallas import tpu_sc as plsc

BATCH = 4
MODEL_DIM = 1024
MAX_LEN = 2048

NC = 2
NS = 16
LANES = 16
NW = NC * NS

B_TOTAL = BATCH * MAX_LEN
T_PER_W = MAX_LEN // NW
CHUNK = 16
QR = 8
QPC = CHUNK // QR
N_TC = T_PER_W // CHUNK
N_CHUNKS = N_TC * BATCH
NB = 5
GLEAD = 3
NQ = MAX_LEN // QR

_VR = MODEL_DIM // LANES

_OFF_A1 = 0
_OFF_A2 = NQ * MODEL_DIM
_OFF_B1 = 2 * NQ * MODEL_DIM
_OFF_B2 = 2 * NQ * MODEL_DIM + QR * MODEL_DIM


def _aux_tables_np():
    i = np.arange(MODEL_DIM, dtype=np.float64)
    w = 1.0 / (10000.0 ** (2.0 * i / MODEL_DIM))
    even = (np.arange(MODEL_DIM) % 2) == 0
    th = QR * np.arange(NQ, dtype=np.float64).reshape(-1, 1) * w
    a1 = np.where(even, np.sin(th), np.cos(th)).astype(np.float32)
    a2 = np.where(even, np.cos(th), -np.sin(th)).astype(np.float32)
    rw = np.arange(QR, dtype=np.float64).reshape(-1, 1) * w
    b1 = np.cos(rw).astype(np.float32)
    b2 = np.sin(rw).astype(np.float32)
    return np.concatenate(
        [a1.reshape(-1), a2.reshape(-1), b1.reshape(-1), b2.reshape(-1)]
    )


_AUX = _aux_tables_np()


def _sc_body(table_hbm, idx_hbm, aux_hbm, out_hbm, *scratch):
    idx_v = scratch[0]
    rows = scratch[1 : 1 + NB]
    posrec = scratch[1 + NB]
    b1_v = scratch[2 + NB]
    b2_v = scratch[3 + NB]
    a_v = scratch[4 + NB : 6 + NB]
    gsem = scratch[6 + NB : 6 + 2 * NB]
    psem = scratch[6 + 2 * NB : 6 + 3 * NB]
    asem = scratch[6 + 3 * NB : 8 + 3 * NB]
    bsem = scratch[8 + 3 * NB]
    isem = scratch[9 + 3 * NB]

    wid = lax.axis_index("s") * NC + lax.axis_index("c")
    t0 = wid * T_PER_W

    bd1 = pltpu.async_copy(aux_hbm.at[pl.ds(_OFF_B1, QR * MODEL_DIM)], b1_v, bsem)
    bd2 = pltpu.async_copy(aux_hbm.at[pl.ds(_OFF_B2, QR * MODEL_DIM)], b2_v, bsem)

    def issue_a(tc):
        q0 = (wid * N_TC + tc) * QPC
        buf = tc % 2
        d1 = pltpu.async_copy(
            aux_hbm.at[pl.ds(_OFF_A1 + q0 * MODEL_DIM, QPC * MODEL_DIM)],
            a_v[buf].at[pl.ds(0, QPC * MODEL_DIM)],
            asem[buf],
        )
        d2 = pltpu.async_copy(
            aux_hbm.at[pl.ds(_OFF_A2 + q0 * MODEL_DIM, QPC * MODEL_DIM)],
            a_v[buf].at[pl.ds(QPC * MODEL_DIM, QPC * MODEL_DIM)],
            asem[buf],
        )
        return (d1, d2)

    a_desc = [None] * N_TC
    a_desc[0] = issue_a(0)
    a_desc[1] = issue_a(1)

    idx_descs = [
        pltpu.async_copy(
            idx_hbm.at[b, pl.ds(t0, T_PER_W)],
            idx_v.at[pl.ds(b * T_PER_W, T_PER_W)],
            isem,
        )
        for b in range(BATCH)
    ]
    for d in idx_descs:
        d.wait()

    def issue_gather(n):
        tc, b = n // BATCH, n % BATCH
        return pltpu.async_copy(
            table_hbm.at[idx_v.at[pl.ds(b * T_PER_W + tc * CHUNK, CHUNK)]],
            rows[n % NB],
            gsem[n % NB],
        )

    gat = [None] * NB
    put = [None] * NB
    for n in range(GLEAD):
        gat[n % NB] = issue_gather(n)

    for c in range(N_CHUNKS):
        j = c % NB
        tc, b = c // BATCH, c % BATCH

        if c == BATCH:
            a_desc[2] = issue_a(2)
        if c == 2 * BATCH:
            a_desc[3] = issue_a(3)

        n = c + GLEAD
        if n < N_CHUNKS:
            if n >= NB:
                put[n % NB].wait()
            gat[n % NB] = issue_gather(n)

        if b == 0:
            if c == 0:
                bd1.wait()
                bd2.wait()
            a_desc[tc][0].wait()
            a_desc[tc][1].wait()
            a_tc = a_v[tc % 2]

            @plsc.parallel_loop(0, _VR, unroll=2)
            def gen_body(jc):
                off = pl.multiple_of(jc << 4, LANES)
                a1q0 = a_tc[pl.ds(off, LANES)]
                a1q1 = a_tc[pl.ds(MODEL_DIM + off, LANES)]
                a2q0 = a_tc[pl.ds(2 * MODEL_DIM + off, LANES)]
                a2q1 = a_tc[pl.ds(3 * MODEL_DIM + off, LANES)]
                for r in range(QR):
                    b1r = b1_v[pl.ds(r * MODEL_DIM + off, LANES)]
                    b2r = b2_v[pl.ds(r * MODEL_DIM + off, LANES)]
                    posrec[r, pl.ds(off, LANES)] = a1q0 * b1r + a2q0 * b2r
                    posrec[QR + r, pl.ds(off, LANES)] = a1q1 * b1r + a2q1 * b2r

        gat[j].wait()
        rows_j = rows[j]

        @plsc.parallel_loop(0, CHUNK * _VR, unroll=8)
        def add_body(i):
            r = i >> 6
            off = pl.multiple_of((i & (_VR - 1)) << 4, LANES)
            rows_j[r, pl.ds(off, LANES)] = (
                rows_j[r, pl.ds(off, LANES)] + posrec[r, pl.ds(off, LANES)]
            )

        put[j] = pltpu.async_copy(
            rows_j,
            out_hbm.at[pl.ds(b * MAX_LEN + t0 + tc * CHUNK, CHUNK)],
            psem[j],
        )

    for c in range(N_CHUNKS - NB, N_CHUNKS):
        put[c % NB].wait()


@jax.jit
def _embed(idx, table, aux):
    mesh = plsc.VectorSubcoreMesh(
        core_axis_name="c", subcore_axis_name="s", num_cores=NC, num_subcores=NS
    )
    scratch = (
        [pltpu.VMEM((BATCH * T_PER_W,), jnp.int32)]
        + [pltpu.VMEM((CHUNK, MODEL_DIM), jnp.float32) for _ in range(NB)]
        + [pltpu.VMEM((CHUNK, MODEL_DIM), jnp.float32)]
        + [pltpu.VMEM((QR * MODEL_DIM,), jnp.float32)]
        + [pltpu.VMEM((QR * MODEL_DIM,), jnp.float32)]
        + [pltpu.VMEM((2 * QPC * MODEL_DIM,), jnp.float32) for _ in range(2)]
        + [pltpu.SemaphoreType.DMA for _ in range(2 * NB + 4)]
    )
    fn = pl.kernel(
        _sc_body,
        out_type=jax.ShapeDtypeStruct((B_TOTAL, MODEL_DIM), jnp.float32),
        mesh=mesh,
        scratch_types=scratch,
    )
    return fn(table, idx, aux)


def kernel(x, table):
    idx = x.astype(jnp.int32)
    out = _embed(idx, table, jnp.asarray(_AUX))
    return out.reshape(BATCH, MAX_LEN, MODEL_DIM)
